# baseline (speedup 1.0000x reference)
import functools

import jax
import jax.numpy as jnp
from jax import lax
from jax.experimental import pallas as pl
from jax.experimental.pallas import tpu as pltpu
from jax.experimental.pallas import tpu_sc as plsc

DIM = 64
NUM_CORES = 2
NUM_SUBCORES = 16
NUM_WORKERS = NUM_CORES * NUM_SUBCORES
CHUNK = 128


def _gather_kernel(n_total):
    b_per_w = n_total // NUM_WORKERS
    n_chunks = b_per_w // CHUNK
    mesh = plsc.VectorSubcoreMesh(
        core_axis_name="c",
        subcore_axis_name="s",
        num_cores=NUM_CORES,
        num_subcores=NUM_SUBCORES,
    )

    @functools.partial(
        pl.kernel,
        out_type=jax.ShapeDtypeStruct((n_total, DIM), jnp.float32),
        mesh=mesh,
        scratch_types=[
            pltpu.VMEM((n_chunks, CHUNK), jnp.int32),
            pltpu.VMEM((2, CHUNK, DIM), jnp.float32),
            pltpu.SemaphoreType.DMA,
            pltpu.SemaphoreType.DMA,
        ],
        compiler_params=pltpu.CompilerParams(use_tc_tiling_on_sc=False),
    )
    def kern(idx_hbm, table_hbm, out_hbm, idx_v, rows_v, gsem, osem):
        wid = lax.axis_index("s") * NUM_CORES + lax.axis_index("c")
        base = wid * b_per_w
        pltpu.sync_copy(idx_hbm.at[wid], idx_v)

        def gather_start(c, buf):
            pltpu.async_copy(table_hbm.at[idx_v.at[c]], rows_v.at[buf], gsem)

        def gather_wait(c, buf):
            pltpu.make_async_copy(
                table_hbm.at[idx_v.at[c]], rows_v.at[buf], gsem
            ).wait()

        def out_start(c, buf):
            pltpu.async_copy(
                rows_v.at[buf], out_hbm.at[pl.ds(base + c * CHUNK, CHUNK)], osem
            )

        def out_wait(c, buf):
            pltpu.make_async_copy(
                rows_v.at[buf], out_hbm.at[pl.ds(base + c * CHUNK, CHUNK)], osem
            ).wait()

        gather_start(0, 0)

        def body(c, _):
            buf = lax.rem(c, 2)
            nxt = 1 - buf
            gather_wait(c, buf)

            @pl.when(c >= 1)
            def _():
                out_wait(c - 1, nxt)

            @pl.when(c + 1 < n_chunks)
            def _():
                gather_start(c + 1, nxt)

            out_start(c, buf)
            return 0

        lax.fori_loop(0, n_chunks, body, 0, unroll=False)
        out_wait(n_chunks - 1, (n_chunks - 1) % 2)

    return kern


def kernel(source, W):
    n_total = source.shape[0] * source.shape[1]
    b_per_w = n_total // NUM_WORKERS
    idx = source.reshape(NUM_WORKERS, b_per_w // CHUNK, CHUNK).astype(jnp.int32)
    out = _gather_kernel(n_total)(idx, W)
    return out.reshape(source.shape[0], source.shape[1], DIM)
